# baseline (speedup 1.0000x reference)
import jax
import jax.numpy as jnp
from jax.experimental import pallas as pl
from jax.experimental.pallas import tpu as pltpu

B, S, D, DEP = 8, 256, 768, 64
T = 4


def _nt(a, w):
    return jax.lax.dot_general(a, w, (((1,), (1,)), ((), ())),
                               preferred_element_type=jnp.float32)


def _tn(a, b):
    return jax.lax.dot_general(a, b, (((0,), (0,)), ((), ())),
                               preferred_element_type=jnp.float32)


def _aspect_kernel(sref, x_ref, dep_h, dr0, dr1, dr2, dr3,
                   wz_h, wf_h, wh_h, bz2, wa, ba2, out_ref,
                   wz_s, wf_s, wh_s, dep_s, sems):
    b = pl.program_id(0)
    a0 = sref[0, b]
    ae = sref[1, b]

    dep_cps = [
        pltpu.make_async_copy(dep_h.at[b, a0 + k], dep_s.at[k], sems.at[3 + k])
        for k in range(T)
    ]
    for c in dep_cps:
        c.start()

    @pl.when(b == 0)
    def _load_params():
        c0 = pltpu.make_async_copy(wz_h, wz_s, sems.at[0])
        c1 = pltpu.make_async_copy(wf_h, wf_s, sems.at[1])
        c2 = pltpu.make_async_copy(wh_h, wh_s, sems.at[2])
        c0.start()
        c1.start()
        c2.start()
        c0.wait()
        c1.wait()
        c2.wait()

    X = x_ref[0]

    wz = wz_s[...]
    wac = wa[:, :D]
    wan = wa[:, D:2 * D]
    wad = wa[:, 2 * D:]
    wfh = wf_s[:, :D]
    wfd = wf_s[:, D:]
    whl = wh_s[:, :D]
    whr = wh_s[:, D:]

    iota0 = jax.lax.broadcasted_iota(jnp.int32, (S, S), 0)
    iota1 = jax.lax.broadcasted_iota(jnp.int32, (S, S), 1)
    strict_lower = (iota0 < iota1).astype(jnp.float32)
    lane = jax.lax.broadcasted_iota(jnp.int32, (1, S), 1)
    subl = jax.lax.broadcasted_iota(jnp.int32, (S, 1), 0)

    wv = jax.lax.dot_general(wz, wan, (((0,), (1,)), ((), ())),
                             preferred_element_type=jnp.float32)
    c0 = jnp.sum(bz2[...] * wan)
    anb_col = jnp.dot(X, wv, preferred_element_type=jnp.float32) + c0

    for c in dep_cps:
        c.wait()
    drs = (dr0, dr1, dr2, dr3)
    oh4 = jnp.concatenate(
        [(lane == a0 + k + 1).astype(jnp.float32) for k in range(T)], axis=0)
    cur_x4 = jnp.dot(oh4, X, preferred_element_type=jnp.float32)
    cur4 = _nt(cur_x4, wz) + bz2[...]
    ba_s = jnp.sum(ba2[...])

    s_rows, m_rows, o1s, o2s = [], [], [], []
    for k in range(T):
        m = (drs[k][0, 0] > 0).astype(jnp.float32)
        m_r = jnp.concatenate([m[:, S - 1:], m[:, :S - 1]], axis=1)
        r1 = jnp.dot(m, strict_lower,
                     preferred_element_type=jnp.float32).astype(jnp.int32)
        r2 = jnp.dot(m_r, strict_lower,
                     preferred_element_type=jnp.float32).astype(jnp.int32)
        o1 = (iota0 == r1).astype(jnp.float32) * m
        o2 = (iota0 == r2).astype(jnp.float32) * m_r
        rank_nb = jnp.dot(o2, anb_col, preferred_element_type=jnp.float32)
        anb_al = _tn(rank_nb, o1)
        adep = jax.lax.dot_general(wad, dep_s[k],
                                   (((1,), (1,)), ((), ())),
                                   preferred_element_type=jnp.float32)
        cs_k = jnp.sum(cur4[k:k + 1] * wac) + ba_s
        s = cs_k + anb_al + adep
        s = jnp.where(s >= 0, s, 0.01 * s)
        s_rows.append(s)
        m_rows.append(m)
        o1s.append(o1)
        o2s.append(o2)

    s4 = jnp.concatenate(s_rows, axis=0)
    m4 = jnp.concatenate(m_rows, axis=0)
    mx = jnp.max(jnp.where(m4 > 0, s4, -1e30), axis=1, keepdims=True)
    e4 = jnp.where(m4 > 0, jnp.exp(s4 - mx), 0.0)
    den = jnp.sum(e4, axis=1, keepdims=True)
    t4 = e4 / den

    u_rows, depsums = [], []
    for k in range(T):
        t_row = t4[k:k + 1]
        depsums.append(jnp.dot(t_row, dep_s[k],
                               preferred_element_type=jnp.float32))
        g = _nt(t_row, o1s[k])
        u_rows.append(jnp.dot(g, o2s[k], preferred_element_type=jnp.float32))

    u4 = jnp.concatenate(u_rows, axis=0)
    depsum4 = jnp.concatenate(depsums, axis=0)
    xsum4 = jnp.dot(u4, X, preferred_element_type=jnp.float32)
    hsum4 = _nt(xsum4, wz) + bz2[...]
    nrep4 = _nt(hsum4, wfh) + _nt(depsum4, wfd)
    temp4 = _nt(nrep4, whl) + _nt(cur4, whr)

    n4 = jnp.sum(m4, axis=1, keepdims=True)
    out = X
    for k in range(T):
        ok = (n4[k, 0] > 0) & (a0 + k <= ae)
        sel = (subl == a0 + k + 1) & ok
        out = jnp.where(sel, temp4[k:k + 1], out)
    out_ref[0] = out


@jax.jit
def kernel(bert_hidden_states, dep_type_adj, text_bert_indices,
           bert_segments_ids, attention_mask, deprel_adj, asp_start, asp_end,
           src_mask, aspect_mask, Wz, bz, Wa, ba, Wf, Wh):
    X = bert_hidden_states
    drel = deprel_adj.reshape(B, S, 1, S)
    scal = jnp.concatenate([asp_start.reshape(1, B), asp_end.reshape(1, B)],
                           axis=0).astype(jnp.int32)
    bz2 = bz.reshape(1, D)
    ba2 = ba.reshape(1, 1)

    def dep_idx(k):
        return lambda b, sref, k=k: (b, sref[0, b] + k, 0, 0)

    full = lambda b, sref: (0, 0)
    hbm = pl.BlockSpec(memory_space=pltpu.MemorySpace.HBM)
    in_specs = [
        pl.BlockSpec((1, S, D), lambda b, sref: (b, 0, 0)),
        hbm,
        *[pl.BlockSpec((1, 1, 1, S), dep_idx(k)) for k in range(T)],
        hbm,
        hbm,
        hbm,
        pl.BlockSpec((1, D), full),
        pl.BlockSpec((1, 2 * D + DEP), full),
        pl.BlockSpec((1, 1), full),
    ]
    grid_spec = pltpu.PrefetchScalarGridSpec(
        num_scalar_prefetch=1,
        grid=(B,),
        in_specs=in_specs,
        out_specs=pl.BlockSpec((1, S, D), lambda b, sref: (b, 0, 0)),
        scratch_shapes=[
            pltpu.VMEM((D, D), jnp.float32),
            pltpu.VMEM((D, D + DEP), jnp.float32),
            pltpu.VMEM((D, 2 * D), jnp.float32),
            pltpu.VMEM((T, S, DEP), jnp.float32),
            pltpu.SemaphoreType.DMA((3 + T,)),
        ],
    )
    out = pl.pallas_call(
        _aspect_kernel,
        grid_spec=grid_spec,
        out_shape=jax.ShapeDtypeStruct((B, S, D), jnp.float32),
    )(scal, X, dep_type_adj,
      drel, drel, drel, drel, Wz, Wf, Wh, bz2, Wa, ba2)
    return out


# speedup vs baseline: 4.3109x; 4.3109x over previous
import jax
import jax.numpy as jnp
from jax.experimental import pallas as pl
from jax.experimental.pallas import tpu as pltpu

B, S, D, DEP = 8, 256, 768, 64
T = 4


def _nt(a, w):
    return jax.lax.dot_general(a, w, (((1,), (1,)), ((), ())),
                               preferred_element_type=jnp.float32)


def _tn(a, b):
    return jax.lax.dot_general(a, b, (((0,), (0,)), ((), ())),
                               preferred_element_type=jnp.float32)


def _aspect_kernel(sref, x_ref, depT_ref, dr0, dr1, dr2, dr3,
                   wz, wfT, wh, bz2, wa, ba2, out_ref):
    b = pl.program_id(0)
    a0 = sref[0, b]
    ae = sref[1, b]
    X = x_ref[0]

    wac = wa[:, :D]
    wan = wa[:, D:2 * D]
    wad = wa[:, 2 * D:]
    whl = wh[:, :D]
    whr = wh[:, D:]

    iota0 = jax.lax.broadcasted_iota(jnp.int32, (S, S), 0)
    iota1 = jax.lax.broadcasted_iota(jnp.int32, (S, S), 1)
    strict_lower = (iota0 < iota1).astype(jnp.float32)
    lane = jax.lax.broadcasted_iota(jnp.int32, (1, S), 1)
    subl = jax.lax.broadcasted_iota(jnp.int32, (S, 1), 0)

    wv = jax.lax.dot_general(wz[...], wan, (((0,), (1,)), ((), ())),
                             preferred_element_type=jnp.float32)
    c0 = jnp.sum(bz2[...] * wan)
    anb_col = jnp.dot(X, wv, preferred_element_type=jnp.float32) + c0

    drs = (dr0, dr1, dr2, dr3)
    oh4 = jnp.concatenate(
        [(lane == a0 + k + 1).astype(jnp.float32) for k in range(T)], axis=0)
    cur_x4 = jnp.dot(oh4, X, preferred_element_type=jnp.float32)
    cur4 = _nt(cur_x4, wz[...]) + bz2[...]
    ba_s = jnp.sum(ba2[...])

    s_rows, m_rows, o1s, o2s = [], [], [], []
    for k in range(T):
        m = (drs[k][0, 0] > 0).astype(jnp.float32)
        m_r = jnp.concatenate([m[:, S - 1:], m[:, :S - 1]], axis=1)
        r1 = jnp.dot(m, strict_lower,
                     preferred_element_type=jnp.float32).astype(jnp.int32)
        r2 = jnp.dot(m_r, strict_lower,
                     preferred_element_type=jnp.float32).astype(jnp.int32)
        o1 = (iota0 == r1).astype(jnp.float32) * m
        o2 = (iota0 == r2).astype(jnp.float32) * m_r
        rank_nb = jnp.dot(o2, anb_col, preferred_element_type=jnp.float32)
        anb_al = _tn(rank_nb, o1)
        adep = jnp.dot(wad, depT_ref[0, k],
                       preferred_element_type=jnp.float32)
        cs_k = jnp.sum(cur4[k:k + 1] * wac) + ba_s
        s = cs_k + anb_al + adep
        s = jnp.where(s >= 0, s, 0.01 * s)
        s_rows.append(s)
        m_rows.append(m)
        o1s.append(o1)
        o2s.append(o2)

    s4 = jnp.concatenate(s_rows, axis=0)
    m4 = jnp.concatenate(m_rows, axis=0)
    mx = jnp.max(jnp.where(m4 > 0, s4, -1e30), axis=1, keepdims=True)
    e4 = jnp.where(m4 > 0, jnp.exp(s4 - mx), 0.0)
    den = jnp.sum(e4, axis=1, keepdims=True)
    t4 = e4 / den

    u_rows, depsums = [], []
    for k in range(T):
        t_row = t4[k:k + 1]
        depsums.append(_nt(t_row, depT_ref[0, k]))
        g = _nt(t_row, o1s[k])
        u_rows.append(jnp.dot(g, o2s[k], preferred_element_type=jnp.float32))

    u4 = jnp.concatenate(u_rows, axis=0)
    depsum4 = jnp.concatenate(depsums, axis=0)
    xsum4 = jnp.dot(u4, X, preferred_element_type=jnp.float32)
    hsum4 = _nt(xsum4, wz[...]) + bz2[...]
    nrep4 = (jnp.dot(hsum4, wfT[:D], preferred_element_type=jnp.float32) +
             jnp.dot(depsum4, wfT[D:], preferred_element_type=jnp.float32))
    temp4 = _nt(nrep4, whl) + _nt(cur4, whr)

    n4 = jnp.sum(m4, axis=1, keepdims=True)
    out = X
    for k in range(T):
        ok = (n4[k, 0] > 0) & (a0 + k <= ae)
        sel = (subl == a0 + k + 1) & ok
        out = jnp.where(sel, temp4[k:k + 1], out)
    out_ref[0] = out


@jax.jit
def kernel(bert_hidden_states, dep_type_adj, text_bert_indices,
           bert_segments_ids, attention_mask, deprel_adj, asp_start, asp_end,
           src_mask, aspect_mask, Wz, bz, Wa, ba, Wf, Wh):
    X = bert_hidden_states
    drel = deprel_adj.reshape(B, S, 1, S)
    scal = jnp.concatenate([asp_start.reshape(1, B), asp_end.reshape(1, B)],
                           axis=0).astype(jnp.int32)
    bz2 = bz.reshape(1, D)
    ba2 = ba.reshape(1, 1)
    WfT = Wf.T

    depT = jnp.swapaxes(dep_type_adj, 2, 3)
    asp_grid = asp_start[:, None] + jnp.arange(T, dtype=asp_start.dtype)
    depT_sl = depT[jnp.arange(B)[:, None], asp_grid]

    def dep_idx(k):
        return lambda b, sref, k=k: (b, sref[0, b] + k, 0, 0)

    full = lambda b, sref: (0, 0)
    in_specs = [
        pl.BlockSpec((1, S, D), lambda b, sref: (b, 0, 0)),
        pl.BlockSpec((1, T, DEP, S), lambda b, sref: (b, 0, 0, 0)),
        *[pl.BlockSpec((1, 1, 1, S), dep_idx(k)) for k in range(T)],
        pl.BlockSpec((D, D), full),
        pl.BlockSpec((D + DEP, D), full),
        pl.BlockSpec((D, 2 * D), full),
        pl.BlockSpec((1, D), full),
        pl.BlockSpec((1, 2 * D + DEP), full),
        pl.BlockSpec((1, 1), full),
    ]
    grid_spec = pltpu.PrefetchScalarGridSpec(
        num_scalar_prefetch=1,
        grid=(B,),
        in_specs=in_specs,
        out_specs=pl.BlockSpec((1, S, D), lambda b, sref: (b, 0, 0)),
    )
    out = pl.pallas_call(
        _aspect_kernel,
        grid_spec=grid_spec,
        out_shape=jax.ShapeDtypeStruct((B, S, D), jnp.float32),
    )(scal, X, depT_sl, drel, drel, drel, drel, Wz, WfT, Wh, bz2, Wa, ba2)
    return out
